# baseline (speedup 1.0000x reference)
import functools

import jax
import jax.numpy as jnp
from jax import lax
from jax.experimental import pallas as pl
from jax.experimental.pallas import tpu as pltpu
from jax.experimental.pallas import tpu_sc as plsc

B = 8
IMG_ROWS, IMG_COLS = 512, 512
P = IMG_ROWS * IMG_COLS
TOT = B * P
NC, NS, LANES = 2, 16, 16
NW = NC * NS
TILE_ROWS = IMG_ROWS // 4
SUB_ROWS = 32
SUB_N = SUB_ROWS * IMG_COLS
M = 16384
X0 = 16.0
SCALE = M / (2.0 * X0)
HL = M + LANES

_mesh = plsc.VectorSubcoreMesh(
    core_axis_name="c", subcore_axis_name="s", num_cores=NC, num_subcores=NS
)
_sc_params = pltpu.CompilerParams(needs_layout_passes=False)


def _stats_body(x_ref, t_ref, o_ref):
    x = x_ref[0]
    t = t_ref[0]
    sp = jnp.maximum(x, 0.0) + jnp.log1p(jnp.exp(-jnp.abs(x)))
    bce_sum = jnp.sum(sp - t * x)
    s = 1.0 / (1.0 + jnp.exp(-x))
    n1 = jnp.sum(t)
    sum1 = jnp.sum(t * (1.0 - s))
    lane = lax.broadcasted_iota(jnp.int32, (1, 1, 128), 2)
    o_ref[...] = (
        jnp.where(lane == 0, bce_sum, 0.0)
        + jnp.where(lane == 1, n1, 0.0)
        + jnp.where(lane == 2, sum1, 0.0)
    )


_stats_call = pl.pallas_call(
    _stats_body,
    grid=(B,),
    in_specs=[
        pl.BlockSpec((1, IMG_ROWS, IMG_COLS), lambda i: (i, 0, 0)),
        pl.BlockSpec((1, IMG_ROWS, IMG_COLS), lambda i: (i, 0, 0)),
    ],
    out_specs=pl.BlockSpec((1, 1, 128), lambda i: (i, 0, 0)),
    out_shape=jax.ShapeDtypeStruct((B, 1, 128), jnp.float32),
)


@functools.partial(
    pl.kernel,
    out_type=jax.ShapeDtypeStruct((NW * HL,), jnp.float32),
    mesh=_mesh,
    compiler_params=_sc_params,
    scratch_types=[
        pltpu.VMEM((SUB_ROWS, IMG_COLS), jnp.float32),
        pltpu.VMEM((SUB_ROWS, IMG_COLS), jnp.float32),
        pltpu.VMEM((SUB_N,), jnp.int32),
        pltpu.VMEM((SUB_N,), jnp.float32),
        pltpu.VMEM((HL,), jnp.float32),
        pltpu.VMEM_SHARED((NS * HL,), jnp.float32),
    ],
)
def _hist_kernel(x_hbm, t_hbm, out_hbm, xb, tb, idxb, onesb, vbuf, hshared):
    sid = lax.axis_index("s")
    wid = lax.axis_index("c") * NS + sid
    img = wid // 4
    row0 = (wid % 4) * TILE_ROWS
    hbase = sid * HL

    def zero_body(j, carry):
        vbuf[pl.ds(j * LANES, LANES)] = jnp.zeros((LANES,), jnp.float32)
        return carry

    lax.fori_loop(0, HL // LANES, zero_body, 0, unroll=4)
    pltpu.sync_copy(vbuf, hshared.at[pl.ds(hbase, HL)])

    def ones_body(j, carry):
        onesb[pl.ds(j * LANES, LANES)] = jnp.ones((LANES,), jnp.float32)
        return carry

    lax.fori_loop(0, SUB_N // LANES, ones_body, 0, unroll=4)

    for sub in range(TILE_ROWS // SUB_ROWS):
        r0 = row0 + sub * SUB_ROWS
        pltpu.sync_copy(x_hbm.at[img, pl.ds(r0, SUB_ROWS), :], xb)
        pltpu.sync_copy(t_hbm.at[img, pl.ds(r0, SUB_ROWS), :], tb)

        def body(r, carry):
            for c in range(IMG_COLS // LANES):
                xv = xb[r, pl.ds(c * LANES, LANES)]
                tv = tb[r, pl.ds(c * LANES, LANES)]
                u = (X0 - xv) * SCALE
                u = jnp.minimum(jnp.maximum(u, 0.0), float(M - 1))
                idx = u.astype(jnp.int32)
                idx = jnp.where(tv == 0.0, idx, M) + hbase
                idxb[pl.ds(r * IMG_COLS + c * LANES, LANES)] = idx
            return carry

        lax.fori_loop(0, SUB_ROWS, body, 0)
        pltpu.sync_copy(onesb, hshared.at[idxb], add=True)

    pltpu.sync_copy(hshared.at[pl.ds(hbase, HL)], vbuf)
    pltpu.sync_copy(vbuf, out_hbm.at[pl.ds(wid * HL, HL)])


@functools.partial(
    pl.kernel,
    out_type=jax.ShapeDtypeStruct((B * LANES,), jnp.float32),
    mesh=_mesh,
    compiler_params=_sc_params,
    scratch_types=[
        pltpu.VMEM((4 * HL,), jnp.float32),
        pltpu.VMEM((128,), jnp.float32),
        pltpu.VMEM((LANES,), jnp.float32),
    ],
)
def _lovasz_kernel(hist_hbm, stats_hbm, out_hbm, hbuf, srow, lbuf):
    cid = lax.axis_index("c")
    sid = lax.axis_index("s")
    img = cid * 4 + sid

    @pl.when(sid < 4)
    def _():
        pltpu.sync_copy(hist_hbm.at[pl.ds(img * 4 * HL, 4 * HL)], hbuf)
        pltpu.sync_copy(stats_hbm.at[img], srow)
        row = srow[pl.ds(0, LANES)]
        lanei = lax.iota(jnp.int32, LANES)
        n1 = jnp.sum(jnp.where(lanei == 1, row, 0.0))
        sum1 = jnp.sum(jnp.where(lanei == 2, row, 0.0))
        lanef = lanei.astype(jnp.float32)

        def body(j, carry):
            cacc, sacc = carry
            base16 = j * LANES
            n16 = (
                hbuf[pl.ds(base16, LANES)]
                + hbuf[pl.ds(HL + base16, LANES)]
                + hbuf[pl.ds(2 * HL + base16, LANES)]
                + hbuf[pl.ds(3 * HL + base16, LANES)]
            )
            chi = cacc + plsc.cumsum(n16)
            clo = chi - n16
            d_w = chi / jnp.maximum(n1 + chi, 1.0) - clo / jnp.maximum(n1 + clo, 1.0)
            xm = X0 - (j.astype(jnp.float32) * LANES + lanef + 0.5) * (1.0 / SCALE)
            sm = 1.0 / (1.0 + jnp.exp(-xm))
            return cacc + jnp.sum(n16), sacc + sm * d_w

        cacc, sacc = lax.fori_loop(
            0, M // LANES, body, (0.0, jnp.zeros((LANES,), jnp.float32)), unroll=4
        )
        lov = (float(P) - n1) * (1.0 / float(P)) + jnp.sum(sacc) + sum1 * (1.0 / float(P))
        lbuf[...] = jnp.full((LANES,), lov, jnp.float32)
        pltpu.sync_copy(lbuf, out_hbm.at[pl.ds(img * LANES, LANES)])


def _combine_body(stats_ref, lov_ref, o_ref):
    lane_s = lax.broadcasted_iota(jnp.int32, (B, 128), 1)
    lane_l = lax.broadcasted_iota(jnp.int32, (B, LANES), 1)
    bce = jnp.sum(jnp.where(lane_s == 0, stats_ref[...], 0.0)) / float(TOT)
    lov = jnp.sum(jnp.where(lane_l == 0, lov_ref[...], 0.0)) / float(B)
    o_ref[...] = jnp.full((1, 1), 0.5 * bce + 0.5 * lov, jnp.float32)


_combine_call = pl.pallas_call(
    _combine_body,
    out_shape=jax.ShapeDtypeStruct((1, 1), jnp.float32),
)


def kernel(input, target):
    x = input.reshape(B, IMG_ROWS, IMG_COLS)
    t = target.reshape(B, IMG_ROWS, IMG_COLS)
    stats = _stats_call(x, t)
    hist = _hist_kernel(x, t)
    lov = _lovasz_kernel(hist, stats.reshape(B, 128))
    out = _combine_call(stats.reshape(B, 128), lov.reshape(B, LANES))
    return out[0, 0]


# speedup vs baseline: 1.3712x; 1.3712x over previous
import functools

import jax
import jax.numpy as jnp
from jax import lax
from jax.experimental import pallas as pl
from jax.experimental.pallas import tpu as pltpu
from jax.experimental.pallas import tpu_sc as plsc

B = 8
IMG_ROWS, IMG_COLS = 512, 512
P = IMG_ROWS * IMG_COLS
TOT = B * P
NC, NS, LANES = 2, 16, 16
NW = NC * NS
TILE_ROWS = IMG_ROWS // 4
SUB_ROWS = 32
M = 16384
X0 = 16.0
SCALE = M / (2.0 * X0)
SENTINEL = -1.0e35

_mesh = plsc.VectorSubcoreMesh(
    core_axis_name="c", subcore_axis_name="s", num_cores=NC, num_subcores=NS
)
_sc_params = pltpu.CompilerParams(needs_layout_passes=False)


def _stats_body(x_ref, t_ref, o_ref, z_ref):
    x = x_ref[0]
    t = t_ref[0]
    sp = jnp.maximum(x, 0.0) + jnp.log1p(jnp.exp(-jnp.abs(x)))
    bce_sum = jnp.sum(sp - t * x)
    s = 1.0 / (1.0 + jnp.exp(-x))
    n1 = jnp.sum(t)
    sum1 = jnp.sum(t * (1.0 - s))
    lane = lax.broadcasted_iota(jnp.int32, (1, 1, 128), 2)
    o_ref[...] = (
        jnp.where(lane == 0, bce_sum, 0.0)
        + jnp.where(lane == 1, n1, 0.0)
        + jnp.where(lane == 2, sum1, 0.0)
    )
    z_ref[0] = jnp.where(t == 1.0, SENTINEL, jnp.maximum(x, -X0))


_stats_call = pl.pallas_call(
    _stats_body,
    grid=(B,),
    in_specs=[
        pl.BlockSpec((1, IMG_ROWS, IMG_COLS), lambda i: (i, 0, 0)),
        pl.BlockSpec((1, IMG_ROWS, IMG_COLS), lambda i: (i, 0, 0)),
    ],
    out_specs=[
        pl.BlockSpec((1, 1, 128), lambda i: (i, 0, 0)),
        pl.BlockSpec((1, IMG_ROWS, IMG_COLS), lambda i: (i, 0, 0)),
    ],
    out_shape=[
        jax.ShapeDtypeStruct((B, 1, 128), jnp.float32),
        jax.ShapeDtypeStruct((B, IMG_ROWS, IMG_COLS), jnp.float32),
    ],
)


@functools.partial(
    pl.kernel,
    out_type=jax.ShapeDtypeStruct((B * LANES,), jnp.float32),
    mesh=_mesh,
    compiler_params=_sc_params,
    scratch_types=[
        pltpu.VMEM((SUB_ROWS, IMG_COLS), jnp.float32),
        pltpu.VMEM((SUB_ROWS, IMG_COLS), jnp.float32),
        pltpu.VMEM((M,), jnp.float32),
        pltpu.VMEM((4 * (M // 2),), jnp.float32),
        pltpu.VMEM((LANES,), jnp.float32),
        pltpu.VMEM((LANES,), jnp.float32),
        pltpu.VMEM_SHARED((NS * M,), jnp.float32),
        pltpu.SemaphoreType.DMA,
        pltpu.SemaphoreType.DMA,
    ],
)
def _lovasz_sc_kernel(
    z_hbm, stats_hbm, out_hbm, zb0, zb1, hist, hbuf, srow, lbuf, hshared, sem0, sem1
):
    sid = lax.axis_index("s")
    wid = lax.axis_index("c") * NS + sid
    img = wid // 4
    row0 = (wid % 4) * TILE_ROWS

    def zero_body(j, carry):
        hist[pl.ds(j * LANES, LANES)] = jnp.zeros((LANES,), jnp.float32)
        return carry

    lax.fori_loop(0, M // LANES, zero_body, 0, unroll=4)

    bufs = (zb0, zb1)
    sems = (sem0, sem1)
    nsub = TILE_ROWS // SUB_ROWS

    def start(c):
        return pltpu.async_copy(
            z_hbm.at[img, pl.ds(row0 + c * SUB_ROWS, SUB_ROWS), :],
            bufs[c % 2],
            sems[c % 2],
        )

    ones = jnp.ones((LANES,), jnp.float32)
    pending = start(0)
    for sub in range(nsub):
        cur = pending
        if sub + 1 < nsub:
            pending = start(sub + 1)
        cur.wait()
        zb = bufs[sub % 2]

        def body(r, carry):
            for c in range(IMG_COLS // LANES):
                zv = zb[r, pl.ds(c * LANES, LANES)]
                u = (X0 - zv) * SCALE
                u = jnp.minimum(jnp.maximum(u, 0.0), float(M - 1))
                idx = u.astype(jnp.int32)
                plsc.addupdate_scatter(hist, [idx], ones, mask=zv > -17.0)
            return carry

        lax.fori_loop(0, SUB_ROWS, body, 0)

    pltpu.sync_copy(hist, hshared.at[pl.ds(sid * M, M)])
    plsc.subcore_barrier()

    @pl.when(sid < 4)
    def _():
        gimg = lax.axis_index("c") * 4 + sid
        pltpu.sync_copy(stats_hbm.at[gimg, pl.ds(0, LANES)], srow)
        row = srow[pl.ds(0, LANES)]
        lanei = lax.iota(jnp.int32, LANES)
        n1 = jnp.sum(jnp.where(lanei == 1, row, 0.0))
        sum1 = jnp.sum(jnp.where(lanei == 2, row, 0.0))
        lanef = lanei.astype(jnp.float32)
        half = M // 2

        carry = (0.0, jnp.zeros((LANES,), jnp.float32))
        for h in range(2):
            for k in range(4):
                pltpu.sync_copy(
                    hshared.at[pl.ds((4 * sid + k) * M + h * half, half)],
                    hbuf.at[pl.ds(k * half, half)],
                )

            def body(j, carry, h=h):
                cacc, sacc = carry
                base16 = j * LANES
                n16 = (
                    hbuf[pl.ds(base16, LANES)]
                    + hbuf[pl.ds(half + base16, LANES)]
                    + hbuf[pl.ds(2 * half + base16, LANES)]
                    + hbuf[pl.ds(3 * half + base16, LANES)]
                )
                chi = cacc + plsc.cumsum(n16)
                clo = chi - n16
                d_w = chi / jnp.maximum(n1 + chi, 1.0) - clo / jnp.maximum(n1 + clo, 1.0)
                bkt = float(h * half) + j.astype(jnp.float32) * LANES + lanef
                xm = X0 - (bkt + 0.5) * (1.0 / SCALE)
                sm = 1.0 / (1.0 + jnp.exp(-xm))
                return cacc + jnp.sum(n16), sacc + sm * d_w

            carry = lax.fori_loop(0, half // LANES, body, carry, unroll=4)
        cacc, sacc = carry
        lov = (float(P) - n1) * (1.0 / float(P)) + jnp.sum(sacc) + sum1 * (1.0 / float(P))
        lbuf[...] = jnp.full((LANES,), lov, jnp.float32)
        pltpu.sync_copy(lbuf, out_hbm.at[pl.ds(gimg * LANES, LANES)])


def _combine_body(stats_ref, lov_ref, o_ref):
    lane_s = lax.broadcasted_iota(jnp.int32, (B, 128), 1)
    lane_l = lax.broadcasted_iota(jnp.int32, (B, LANES), 1)
    bce = jnp.sum(jnp.where(lane_s == 0, stats_ref[...], 0.0)) / float(TOT)
    lov = jnp.sum(jnp.where(lane_l == 0, lov_ref[...], 0.0)) / float(B)
    o_ref[...] = jnp.full((1, 1), 0.5 * bce + 0.5 * lov, jnp.float32)


_combine_call = pl.pallas_call(
    _combine_body,
    out_shape=jax.ShapeDtypeStruct((1, 1), jnp.float32),
)


def kernel(input, target):
    x = input.reshape(B, IMG_ROWS, IMG_COLS)
    t = target.reshape(B, IMG_ROWS, IMG_COLS)
    stats, z = _stats_call(x, t)
    stats2 = stats.reshape(B, 128)
    lov = _lovasz_sc_kernel(z, stats2)
    out = _combine_call(stats2, lov.reshape(B, LANES))
    return out[0, 0]


# speedup vs baseline: 2.0257x; 1.4773x over previous
import functools

import jax
import jax.numpy as jnp
from jax import lax
from jax.experimental import pallas as pl
from jax.experimental.pallas import tpu as pltpu
from jax.experimental.pallas import tpu_sc as plsc

B = 8
IMG_ROWS, IMG_COLS = 512, 512
P = IMG_ROWS * IMG_COLS
TOT = B * P
NC, NS, LANES = 2, 16, 16
NW = NC * NS
TILE_ROWS = IMG_ROWS // 4
SUB_ROWS = 64
M = 16384
X0 = 16.0
SCALE = M / (2.0 * X0)
SENTINEL = -1.0e35

_mesh = plsc.VectorSubcoreMesh(
    core_axis_name="c", subcore_axis_name="s", num_cores=NC, num_subcores=NS
)
_sc_params = pltpu.CompilerParams(needs_layout_passes=False)


def _stats_body(x_ref, t_ref, o_ref, z_ref):
    x = x_ref[0]
    t = t_ref[0]
    sp = jnp.maximum(x, 0.0) + jnp.log1p(jnp.exp(-jnp.abs(x)))
    bce_sum = jnp.sum(sp - t * x)
    s = 1.0 / (1.0 + jnp.exp(-x))
    n1 = jnp.sum(t)
    sum1 = jnp.sum(t * (1.0 - s))
    lane = lax.broadcasted_iota(jnp.int32, (1, 1, 128), 2)
    o_ref[...] = (
        jnp.where(lane == 0, bce_sum, 0.0)
        + jnp.where(lane == 1, n1, 0.0)
        + jnp.where(lane == 2, sum1, 0.0)
    )
    u = (X0 - x) * SCALE
    u = jnp.minimum(jnp.maximum(u, 0.0), float(M - 1))
    idx = u.astype(jnp.int32)
    z_ref[0] = jnp.where(t == 1.0, M, idx).astype(jnp.int16)


_stats_call = pl.pallas_call(
    _stats_body,
    grid=(B,),
    in_specs=[
        pl.BlockSpec((1, IMG_ROWS, IMG_COLS), lambda i: (i, 0, 0)),
        pl.BlockSpec((1, IMG_ROWS, IMG_COLS), lambda i: (i, 0, 0)),
    ],
    out_specs=[
        pl.BlockSpec((1, 1, 128), lambda i: (i, 0, 0)),
        pl.BlockSpec((1, IMG_ROWS, IMG_COLS), lambda i: (i, 0, 0)),
    ],
    out_shape=[
        jax.ShapeDtypeStruct((B, 1, 128), jnp.float32),
        jax.ShapeDtypeStruct((B, IMG_ROWS, IMG_COLS), jnp.int16),
    ],
)


@functools.partial(
    pl.kernel,
    out_type=jax.ShapeDtypeStruct((B * LANES,), jnp.float32),
    mesh=_mesh,
    compiler_params=_sc_params,
    scratch_types=[
        pltpu.VMEM((SUB_ROWS, IMG_COLS), jnp.int16),
        pltpu.VMEM((SUB_ROWS, IMG_COLS), jnp.int16),
        pltpu.VMEM((M,), jnp.float32),
        pltpu.VMEM((4 * (M // 2),), jnp.float32),
        pltpu.VMEM((LANES,), jnp.float32),
        pltpu.VMEM((LANES,), jnp.float32),
        pltpu.VMEM_SHARED((NS * M,), jnp.float32),
        pltpu.SemaphoreType.DMA,
        pltpu.SemaphoreType.DMA,
    ],
)
def _lovasz_sc_kernel(
    z_hbm, stats_hbm, out_hbm, zb0, zb1, hist, hbuf, srow, lbuf, hshared, sem0, sem1
):
    sid = lax.axis_index("s")
    wid = lax.axis_index("c") * NS + sid
    img = wid // 4
    row0 = (wid % 4) * TILE_ROWS

    def zero_body(j, carry):
        hist[pl.ds(j * LANES, LANES)] = jnp.zeros((LANES,), jnp.float32)
        return carry

    lax.fori_loop(0, M // LANES, zero_body, 0, unroll=4)

    bufs = (zb0, zb1)
    sems = (sem0, sem1)
    nsub = TILE_ROWS // SUB_ROWS

    def start(c):
        return pltpu.async_copy(
            z_hbm.at[img, pl.ds(row0 + c * SUB_ROWS, SUB_ROWS), :],
            bufs[c % 2],
            sems[c % 2],
        )

    ones = jnp.ones((LANES,), jnp.float32)
    pending = start(0)
    for sub in range(nsub):
        cur = pending
        if sub + 1 < nsub:
            pending = start(sub + 1)
        cur.wait()
        zb = bufs[sub % 2]

        def body(r, carry):
            for c in range(IMG_COLS // (2 * LANES)):
                v32 = zb[r, pl.ds(c * 2 * LANES, 2 * LANES)]
                ia, ib = plsc.unpack(v32, format=plsc.PackFormat.INTERLEAVED)
                plsc.addupdate_scatter(hist, [ia], ones, mask=ia < M)
                plsc.addupdate_scatter(hist, [ib], ones, mask=ib < M)
            return carry

        lax.fori_loop(0, SUB_ROWS, body, 0)

    pltpu.sync_copy(hist, hshared.at[pl.ds(sid * M, M)])
    plsc.subcore_barrier()

    @pl.when(sid < 4)
    def _():
        gimg = lax.axis_index("c") * 4 + sid
        pltpu.sync_copy(stats_hbm.at[gimg, pl.ds(0, LANES)], srow)
        row = srow[pl.ds(0, LANES)]
        lanei = lax.iota(jnp.int32, LANES)
        n1 = jnp.sum(jnp.where(lanei == 1, row, 0.0))
        sum1 = jnp.sum(jnp.where(lanei == 2, row, 0.0))
        lanef = lanei.astype(jnp.float32)
        half = M // 2

        carry = (0.0, jnp.zeros((LANES,), jnp.float32))
        for h in range(2):
            for k in range(4):
                pltpu.sync_copy(
                    hshared.at[pl.ds((4 * sid + k) * M + h * half, half)],
                    hbuf.at[pl.ds(k * half, half)],
                )

            def body(j, carry, h=h):
                cacc, sacc = carry
                base16 = j * LANES
                n16 = (
                    hbuf[pl.ds(base16, LANES)]
                    + hbuf[pl.ds(half + base16, LANES)]
                    + hbuf[pl.ds(2 * half + base16, LANES)]
                    + hbuf[pl.ds(3 * half + base16, LANES)]
                )
                chi = cacc + plsc.cumsum(n16)
                clo = chi - n16
                d_w = chi / jnp.maximum(n1 + chi, 1.0) - clo / jnp.maximum(n1 + clo, 1.0)
                bkt = float(h * half) + j.astype(jnp.float32) * LANES + lanef
                xm = X0 - (bkt + 0.5) * (1.0 / SCALE)
                sm = 1.0 / (1.0 + jnp.exp(-xm))
                return cacc + jnp.sum(n16), sacc + sm * d_w

            carry = lax.fori_loop(0, half // LANES, body, carry, unroll=4)
        cacc, sacc = carry
        lov = (float(P) - n1) * (1.0 / float(P)) + jnp.sum(sacc) + sum1 * (1.0 / float(P))
        lbuf[...] = jnp.full((LANES,), lov, jnp.float32)
        pltpu.sync_copy(lbuf, out_hbm.at[pl.ds(gimg * LANES, LANES)])


def _combine_body(stats_ref, lov_ref, o_ref):
    lane_s = lax.broadcasted_iota(jnp.int32, (B, 128), 1)
    lane_l = lax.broadcasted_iota(jnp.int32, (B, LANES), 1)
    bce = jnp.sum(jnp.where(lane_s == 0, stats_ref[...], 0.0)) / float(TOT)
    lov = jnp.sum(jnp.where(lane_l == 0, lov_ref[...], 0.0)) / float(B)
    o_ref[...] = jnp.full((1, 1), 0.5 * bce + 0.5 * lov, jnp.float32)


_combine_call = pl.pallas_call(
    _combine_body,
    out_shape=jax.ShapeDtypeStruct((1, 1), jnp.float32),
)


def kernel(input, target):
    x = input.reshape(B, IMG_ROWS, IMG_COLS)
    t = target.reshape(B, IMG_ROWS, IMG_COLS)
    stats, z = _stats_call(x, t)
    stats2 = stats.reshape(B, 128)
    lov = _lovasz_sc_kernel(z, stats2)
    out = _combine_call(stats2, lov.reshape(B, LANES))
    return out[0, 0]
